# baseline (speedup 1.0000x reference)
import numpy as np
import jax
import jax.numpy as jnp
from jax.experimental import pallas as pl

DIM = 20
BBLK = 2048


def _dotb(a, b):
    return jax.lax.dot_general(
        a, b, (((1,), (0,)), ((), ())),
        precision=jax.lax.Precision.DEFAULT,
        preferred_element_type=jnp.float32)


_MD = np.zeros((DIM, DIM, 1), np.float32)
_MS = np.zeros((DIM, DIM, 1), np.float32)
_MP = np.zeros((DIM, DIM, 1), np.float32)
for _i in range(DIM):
    _MD[_i, _i, 0] = 1.0
    _MS[_i, (_i - 1) % DIM, 0] = 1.0
    _MP[_i, (_i + 1) % DIM, 0] = 1.0


def _fwd_kernel(x_ref, g1w_ref, b1_ref, w2_ref, b2_ref, w3_ref, b3_ref,
                w4_ref, b4_ref, md_ref, ms_ref, mp_ref, f1_ref, g1_ref,
                f2_ref, g2_ref):
    x = x_ref[...]
    h1 = jnp.maximum(_dotb(x, g1w_ref[...]) + b1_ref[...],
                     0.0)
    w2 = w2_ref[...]
    w3 = w3_ref[...]
    b2 = b2_ref[...]
    b3 = b3_ref[...]
    h3_parts = []
    for kt in range(5):
        h1k = h1[:, 128 * kt:128 * kt + 128]
        h2k = jnp.maximum(_dotb(h1k, w2) + b2, 0.0)
        h3k = jnp.maximum(_dotb(h2k, w3) + b3, 0.0)
        h3_parts.append(h3k)
    h3 = jnp.concatenate(h3_parts, axis=1)
    o = _dotb(h3, w4_ref[...]) + b4_ref[...]
    ot = o.T

    f1_ref[...] = ot[0:20, :][:, None, :]
    f2_ref[...] = ot[60:80, :][:, None, :]

    md = md_ref[...]
    ms = ms_ref[...]
    mp = mp_ref[...]

    sa = ot[20:40, :][None, :, :]
    da = ot[40:60, :][None, :, :]
    g1_ref[...] = md * da + ms * sa

    sb = ot[80:100, :][None, :, :]
    db = ot[100:120, :][None, :, :]
    pb = ot[120:140, :][None, :, :]
    g2_ref[...] = md * db + (ms * sb + mp * pb)


def _blockdiag2(A, B):
    za = jnp.zeros((A.shape[0], B.shape[1]), A.dtype)
    zb = jnp.zeros((B.shape[0], A.shape[1]), A.dtype)
    return jnp.concatenate(
        [jnp.concatenate([A, za], axis=1),
         jnp.concatenate([zb, B], axis=1)], axis=0)


_M2 = np.zeros((3, DIM, DIM * 32), np.float32)
for _t in range(3):
    for _i in range(DIM):
        _M2[_t, (_i + _t - 1) % DIM, 32 * _i:32 * _i + 32] = 1.0
_E2 = np.kron(np.eye(4, dtype=np.float32), np.ones((32, 64), np.float32))
_E3 = np.kron(np.eye(4, dtype=np.float32), np.ones((64, 32), np.float32))
_P2 = np.zeros((DIM * 32, 7 * DIM), np.float32)
_sub = lambda n: (n - 1) % DIM
_sup = lambda n: (n + 1) % DIM
_gcols = [lambda n: n, _sub, lambda n: n, lambda n: n, _sub, lambda n: n, _sup]
for _g in range(7):
    for _n in range(DIM):
        _P2[32 * _n:32 * _n + 32, DIM * _g + _gcols[_g](_n)] = 1.0


def _build_kernel(Wa0_ref, ba0_ref, Wa1_ref, ba1_ref, Wa2_ref, ba2_ref,
                  Wa3_ref, ba3_ref, Wb0_ref, bb0_ref, Wb1_ref, bb1_ref,
                  Wb2_ref, bb2_ref, Wb3_ref, bb3_ref, m2_ref, e2_ref,
                  e3_ref, p2_ref, g1w_ref, b1_ref, w2_ref, b2_ref, w3_ref,
                  b3_ref, w4_ref, b4_ref):
    cat = jnp.concatenate
    W0c = cat([Wa0_ref[...],
               cat([jnp.zeros((1, 16), jnp.float32), Wb0_ref[...]], axis=0)],
              axis=1)
    W0r = cat([W0c] * DIM, axis=1)
    g1w_ref[...] = (m2_ref[...] * W0r[:, None, :]).sum(0)
    b0c = cat([ba0_ref[...], bb0_ref[...]], axis=1)
    b1_ref[...] = cat([b0c] * DIM, axis=1)

    W1c = _blockdiag2(Wa1_ref[...], Wb1_ref[...])
    W1r = cat([cat([W1c] * 4, axis=1)] * 4, axis=0)
    w2_ref[...] = e2_ref[...] * W1r
    b1c = cat([ba1_ref[...], bb1_ref[...]], axis=1)
    b2_ref[...] = cat([b1c] * 4, axis=1)

    W2c = _blockdiag2(Wa2_ref[...], Wb2_ref[...])
    W2r = cat([cat([W2c] * 4, axis=1)] * 4, axis=0)
    w3_ref[...] = e3_ref[...] * W2r
    b2c = cat([ba2_ref[...], bb2_ref[...]], axis=1)
    b3_ref[...] = cat([b2c] * 4, axis=1)

    W3c = _blockdiag2(Wa3_ref[...], Wb3_ref[...])
    W3g = cat([jnp.broadcast_to(W3c[:, g:g + 1], (32, DIM))
               for g in range(7)], axis=1)
    W3r = cat([W3g] * DIM, axis=0)
    w4_ref[...] = p2_ref[...] * W3r
    b3c = cat([ba3_ref[...], bb3_ref[...]], axis=1)
    b4_ref[...] = cat([jnp.broadcast_to(b3c[:, g:g + 1], (1, DIM))
                       for g in range(7)], axis=1)


def kernel(x, Wa0, ba0, Wa1, ba1, Wa2, ba2, Wa3, ba3,
           Wb0, bb0, Wb1, bb1, Wb2, bb2, Wb3, bb3):
    batch = x.shape[0]
    full = lambda shape: pl.BlockSpec(shape, lambda b: (0,) * len(shape))
    fullw = lambda shape: pl.BlockSpec(shape, lambda: (0,) * len(shape))

    raw = [Wa0, ba0.reshape(1, -1), Wa1, ba1.reshape(1, -1),
           Wa2, ba2.reshape(1, -1), Wa3, ba3.reshape(1, -1),
           Wb0, bb0.reshape(1, -1), Wb1, bb1.reshape(1, -1),
           Wb2, bb2.reshape(1, -1), Wb3, bb3.reshape(1, -1),
           jnp.asarray(_M2), jnp.asarray(_E2), jnp.asarray(_E3),
           jnp.asarray(_P2)]
    G1, B1, W2chunk, B2, W3chunk, B3, W4, B4 = pl.pallas_call(
        _build_kernel,
        in_specs=[fullw(a.shape) for a in raw],
        out_specs=[
            fullw((DIM, DIM * 32)), fullw((1, DIM * 32)),
            fullw((128, 256)), fullw((1, 256)),
            fullw((256, 128)), fullw((1, 128)),
            fullw((DIM * 32, 7 * DIM)), fullw((1, 7 * DIM)),
        ],
        out_shape=[
            jax.ShapeDtypeStruct((DIM, DIM * 32), jnp.float32),
            jax.ShapeDtypeStruct((1, DIM * 32), jnp.float32),
            jax.ShapeDtypeStruct((128, 256), jnp.float32),
            jax.ShapeDtypeStruct((1, 256), jnp.float32),
            jax.ShapeDtypeStruct((256, 128), jnp.float32),
            jax.ShapeDtypeStruct((1, 128), jnp.float32),
            jax.ShapeDtypeStruct((DIM * 32, 7 * DIM), jnp.float32),
            jax.ShapeDtypeStruct((1, 7 * DIM), jnp.float32),
        ],
    )(*raw)

    grid = (batch // BBLK,)
    f1t, g1t, f2t, g2t = pl.pallas_call(
        _fwd_kernel,
        grid=grid,
        in_specs=[
            pl.BlockSpec((BBLK, DIM), lambda b: (b, 0)),
            full(G1.shape), full(B1.shape),
            full(W2chunk.shape), full(B2.shape),
            full(W3chunk.shape), full(B3.shape),
            full(W4.shape), full(B4.shape),
            full((DIM, DIM, 1)), full((DIM, DIM, 1)), full((DIM, DIM, 1)),
        ],
        out_specs=[
            pl.BlockSpec((DIM, 1, BBLK), lambda b: (0, 0, b)),
            pl.BlockSpec((DIM, DIM, BBLK), lambda b: (0, 0, b)),
            pl.BlockSpec((DIM, 1, BBLK), lambda b: (0, 0, b)),
            pl.BlockSpec((DIM, DIM, BBLK), lambda b: (0, 0, b)),
        ],
        out_shape=[
            jax.ShapeDtypeStruct((DIM, 1, batch), jnp.float32),
            jax.ShapeDtypeStruct((DIM, DIM, batch), jnp.float32),
            jax.ShapeDtypeStruct((DIM, 1, batch), jnp.float32),
            jax.ShapeDtypeStruct((DIM, DIM, batch), jnp.float32),
        ],
    )(x, G1, B1, W2chunk, B2, W3chunk, B3, W4, B4,
      jnp.asarray(_MD), jnp.asarray(_MS), jnp.asarray(_MP))
    f1 = jnp.transpose(f1t, (2, 0, 1))
    f2 = jnp.transpose(f2t, (2, 0, 1))
    g1 = jnp.transpose(g1t, (2, 0, 1))
    g2 = jnp.transpose(g2t, (2, 0, 1))
    return (f1, g1, f2, g2)
